# Initial kernel scaffold; baseline (speedup 1.0000x reference)
import jax
import jax.numpy as jnp
from jax.experimental import pallas as pl


def kernel(input, mlp1, mlp2, mlp3, cls):
    raise NotImplementedError("write your pallas kernel here")


# speedup vs baseline: 2.1899x; 2.1899x over previous
import functools

import jax
import jax.numpy as jnp
from jax.experimental import pallas as pl


def _fps_body(xyz_ref, out_ref, *, npoint, n):
    x = xyz_ref[:, 0, :]
    y = xyz_ref[:, 1, :]
    z = xyz_ref[:, 2, :]
    b = x.shape[0]
    lane = (jax.lax.broadcasted_iota(jnp.int32, (b, n), 1)
            + jax.lax.broadcasted_iota(jnp.int32, (b, n), 0) * 0)
    step_iota = (jax.lax.broadcasted_iota(jnp.int32, (b, npoint), 1)
                 + jax.lax.broadcasted_iota(jnp.int32, (b, npoint), 0) * 0)

    out_ref[...] = step_iota * 0

    def step(i, carry):
        dists, far = carry
        farb = far + step_iota * 0
        out_ref[...] = jnp.where(step_iota == i, farb, out_ref[...])
        sel = lane == far
        cx = jnp.sum(jnp.where(sel, x, 0.0), axis=1, keepdims=True)
        cy = jnp.sum(jnp.where(sel, y, 0.0), axis=1, keepdims=True)
        cz = jnp.sum(jnp.where(sel, z, 0.0), axis=1, keepdims=True)
        d = (x - cx) ** 2 + (y - cy) ** 2 + (z - cz) ** 2
        dists = jnp.minimum(dists, d)
        m = jnp.max(dists, axis=1, keepdims=True)
        far = jnp.min(jnp.where(dists == m, lane, n), axis=1, keepdims=True)
        return dists, far

    dists0 = x * 0.0 + 1e10
    far0 = lane[:, 0:1] * 0
    jax.lax.fori_loop(0, npoint, step, (dists0, far0))


def _fps(xyz, npoint):
    b, n, _ = xyz.shape
    xyz_t = jnp.transpose(xyz, (0, 2, 1))
    return pl.pallas_call(
        functools.partial(_fps_body, npoint=npoint, n=n),
        out_shape=jax.ShapeDtypeStruct((b, npoint), jnp.int32),
    )(xyz_t)


def _bq_body(pts_ref, cent_ref, out_ref, *, radius, nsample, n, s):
    px = pts_ref[0, :, 0:1]
    py = pts_ref[0, :, 1:2]
    pz = pts_ref[0, :, 2:3]
    cx = cent_ref[0, 0:1, :]
    cy = cent_ref[0, 1:2, :]
    cz = cent_ref[0, 2:3, :]
    d2 = (px - cx) ** 2 + (py - cy) ** 2 + (pz - cz) ** 2
    idx = (jax.lax.broadcasted_iota(jnp.int32, (n, s), 0)
           + jax.lax.broadcasted_iota(jnp.int32, (n, s), 1) * 0)
    cand = jnp.where(d2 < radius * radius, idx, n)
    rows = []
    for _ in range(nsample):
        cur = jnp.min(cand, axis=0, keepdims=True)
        rows.append(cur)
        cand = jnp.where(cand == cur, n, cand)
    outm = jnp.concatenate(rows, axis=0)
    outm = jnp.where(outm == n, outm[0:1, :], outm)
    out_ref[0] = outm


def _ball_query(xyz, new_xyz, radius, nsample):
    b, n, _ = xyz.shape
    s = new_xyz.shape[1]
    cent_t = jnp.transpose(new_xyz, (0, 2, 1))
    out = pl.pallas_call(
        functools.partial(_bq_body, radius=radius, nsample=nsample, n=n, s=s),
        grid=(b,),
        in_specs=[
            pl.BlockSpec((1, n, 3), lambda i: (i, 0, 0)),
            pl.BlockSpec((1, 3, s), lambda i: (i, 0, 0)),
        ],
        out_specs=pl.BlockSpec((1, nsample, s), lambda i: (i, 0, 0)),
        out_shape=jax.ShapeDtypeStruct((b, nsample, s), jnp.int32),
    )(xyz, cent_t)
    return jnp.transpose(out, (0, 2, 1))


def _layer_body(x_ref, a_ref, c_ref, w_ref, y_ref, p_ref, *, fuse):
    x = x_ref[...]
    if fuse:
        x = jnp.maximum(x * a_ref[...] + c_ref[...], 0.0)
    y = jnp.dot(x, w_ref[...], preferred_element_type=jnp.float32)
    y_ref[...] = y
    p_ref[0, 0, :] = jnp.sum(y, axis=0)
    p_ref[0, 1, :] = jnp.sum(y * y, axis=0)


def _layer(x, a, c, wt, fuse, tile):
    r, cin = x.shape
    cout = wt.shape[1]
    ncell = r // tile
    y, p = pl.pallas_call(
        functools.partial(_layer_body, fuse=fuse),
        grid=(ncell,),
        in_specs=[
            pl.BlockSpec((tile, cin), lambda i: (i, 0)),
            pl.BlockSpec((1, cin), lambda i: (0, 0)),
            pl.BlockSpec((1, cin), lambda i: (0, 0)),
            pl.BlockSpec((cin, cout), lambda i: (0, 0)),
        ],
        out_specs=[
            pl.BlockSpec((tile, cout), lambda i: (i, 0)),
            pl.BlockSpec((1, 2, cout), lambda i: (i, 0, 0)),
        ],
        out_shape=[
            jax.ShapeDtypeStruct((r, cout), jnp.float32),
            jax.ShapeDtypeStruct((ncell, 2, cout), jnp.float32),
        ],
    )(x, a.reshape(1, cin), c.reshape(1, cin), wt)
    return y, p


def _maxpool_body(y_ref, a_ref, c_ref, o_ref):
    a = a_ref[...]
    c = c_ref[...]
    ns = y_ref.shape[1]
    m = jnp.maximum(y_ref[:, 0, :] * a + c, 0.0)
    for k in range(1, ns):
        m = jnp.maximum(m, jnp.maximum(y_ref[:, k, :] * a + c, 0.0))
    o_ref[...] = m


def _maxpool(y, a, c, groups, ns, gtile):
    cout = y.shape[-1]
    yg = y.reshape(groups, ns, cout)
    return pl.pallas_call(
        _maxpool_body,
        grid=(groups // gtile,),
        in_specs=[
            pl.BlockSpec((gtile, ns, cout), lambda i: (i, 0, 0)),
            pl.BlockSpec((1, cout), lambda i: (0, 0)),
            pl.BlockSpec((1, cout), lambda i: (0, 0)),
        ],
        out_specs=pl.BlockSpec((gtile, cout), lambda i: (i, 0)),
        out_shape=jax.ShapeDtypeStruct((groups, cout), jnp.float32),
    )(yg, a.reshape(1, cout), c.reshape(1, cout))


def _mlp_stack(x, params, b, s, ns, tile, gtile):
    r = x.shape[0]
    a = None
    c = None
    y = x
    for li, (w, g, bb) in enumerate(params):
        wt = jnp.transpose(w)
        y, p = _layer(y, a if a is not None else jnp.zeros((y.shape[1],), jnp.float32),
                      c if c is not None else jnp.zeros((y.shape[1],), jnp.float32),
                      wt, fuse=li > 0, tile=tile)
        ps = jnp.sum(p, axis=0)
        mean = ps[0] / r
        var = ps[1] / r - mean * mean
        a = g / jnp.sqrt(var + 1e-5)
        c = bb - mean * a
    cout = y.shape[-1]
    f = _maxpool(y, a, c, b * s, ns, gtile)
    return f.reshape(b, s, cout)


def _cls_body(x_ref, w1_ref, b1_ref, w2_ref, b2_ref, w3_ref, b3_ref, o_ref):
    h = jnp.maximum(
        jnp.dot(x_ref[...], w1_ref[...], preferred_element_type=jnp.float32)
        + b1_ref[...], 0.0)
    h = jnp.maximum(
        jnp.dot(h, w2_ref[...], preferred_element_type=jnp.float32)
        + b2_ref[...], 0.0)
    o_ref[...] = (
        jnp.dot(h, w3_ref[...], preferred_element_type=jnp.float32)
        + b3_ref[...])


def _classifier(x, cls):
    b = x.shape[0]
    (w1, b1), (w2, b2), (w3, b3) = cls
    return pl.pallas_call(
        _cls_body,
        out_shape=jax.ShapeDtypeStruct((b, w3.shape[0]), jnp.float32),
    )(x, jnp.transpose(w1), b1.reshape(1, -1),
      jnp.transpose(w2), b2.reshape(1, -1),
      jnp.transpose(w3), b3.reshape(1, -1))


def _gather(points, idx):
    b, s, ns = idx.shape
    flat = idx.reshape(b, s * ns)
    out = jnp.take_along_axis(points, flat[:, :, None], axis=1)
    return out.reshape(b, s, ns, points.shape[-1])


def kernel(input, mlp1, mlp2, mlp3, cls):
    b, n, _ = input.shape
    xyz = input[:, :, :3]

    fidx = _fps(xyz, 512)
    new_xyz = jnp.take_along_axis(xyz, fidx[:, :, None], axis=1)
    gidx = _ball_query(xyz, new_xyz, 0.2, 32)
    gx = _gather(xyz, gidx) - new_xyz[:, :, None, :]
    x1 = gx.reshape(b * 512 * 32, 3)
    f1 = _mlp_stack(x1, mlp1, b, 512, 32, tile=2048, gtile=64)

    fidx2 = _fps(new_xyz, 128)
    xyz2 = jnp.take_along_axis(new_xyz, fidx2[:, :, None], axis=1)
    gidx2 = _ball_query(new_xyz, xyz2, 0.4, 64)
    gx2 = _gather(new_xyz, gidx2) - xyz2[:, :, None, :]
    gf2 = _gather(f1, gidx2)
    x2 = jnp.concatenate([gx2, gf2], axis=-1).reshape(b * 128 * 64, 131)
    f2 = _mlp_stack(x2, mlp2, b, 128, 64, tile=2048, gtile=64)

    x3 = jnp.concatenate([xyz2[:, None, :, :], f2[:, None, :, :]],
                         axis=-1).reshape(b * 128, 259)
    f3 = _mlp_stack(x3, mlp3, b, 1, 128, tile=2048, gtile=16)

    return _classifier(f3[:, 0, :], cls)
